# baseline (speedup 1.0000x reference)
import functools

import jax
import jax.numpy as jnp
from jax import lax
from jax.experimental import pallas as pl
from jax.experimental.pallas import tpu as pltpu
from jax.experimental.pallas import tpu_sc as plsc


def _router_kernel(T, NB, x_ref, gw_ref, gb_ref,
                   rank_ref, be_ref, bv_ref,
                   r1_ref, r2_ref, wn1_ref, wn2_ref):
    S, Hd = x_ref.shape
    E = gw_ref.shape[1]
    f32 = jnp.float32
    bf16 = jnp.bfloat16

    logits = lax.dot(x_ref[...], gw_ref[...].astype(bf16),
                     preferred_element_type=f32) + gb_ref[...]

    m1 = jnp.max(logits, axis=1, keepdims=True)
    ex = jnp.exp(logits - m1)
    z = jnp.sum(ex, axis=1, keepdims=True)
    p = ex / z
    ioe = lax.broadcasted_iota(jnp.int32, (S, E), 1)
    pm1 = jnp.max(p, axis=1, keepdims=True)
    i1 = jnp.min(jnp.where(p == pm1, ioe, E), axis=1, keepdims=True)
    p_m = jnp.where(ioe == i1, -1.0, p)
    pm2 = jnp.max(p_m, axis=1, keepdims=True)
    i2 = jnp.min(jnp.where(p_m == pm2, ioe, E), axis=1, keepdims=True)
    denom = pm1 + pm2 + 1e-6
    wn1_ref[...] = jnp.broadcast_to(pm1 / denom, wn1_ref.shape)
    wn2_ref[...] = jnp.broadcast_to(pm2 / denom, wn2_ref.shape)
    sel1 = ioe == i1
    sel2 = ioe == i2
    m = jnp.logical_or(sel1, sel2).astype(f32)

    rio = lax.broadcasted_iota(jnp.int32, (S, S), 0)
    cio = lax.broadcasted_iota(jnp.int32, (S, S), 1)
    ltri = (cio < rio).astype(bf16)
    rank = lax.dot(ltri, m.astype(bf16), preferred_element_type=f32)
    rank_ref[...] = jnp.where(m > 0.5, rank, -1.0)

    ones_col = jnp.ones((S, 1), bf16)
    counts_t = lax.dot_general(m.astype(bf16), ones_col,
                               (((0,), (0,)), ((), ())),
                               preferred_element_type=f32)
    nb_t = jnp.floor((counts_t + (T - 1)) / T)
    eio_r = lax.broadcasted_iota(jnp.int32, (E, E), 0)
    eio_c = lax.broadcasted_iota(jnp.int32, (E, E), 1)
    l8 = (eio_r > eio_c).astype(bf16)
    pad_t = lax.dot(l8, nb_t.astype(bf16), preferred_element_type=f32)
    total = jnp.sum(nb_t, axis=0, keepdims=True)

    counts_row = jnp.sum(m, axis=0, keepdims=True)
    nb_row = jnp.floor((counts_row + (T - 1)) / T)
    l8u = (eio_r < eio_c).astype(bf16)
    pad_row = lax.dot(nb_row.astype(bf16), l8u,
                      preferred_element_type=f32)
    base_row = pad_row * T

    r1v = jnp.sum(jnp.where(sel1, rank + base_row, 0.0), axis=1, keepdims=True)
    r2v = jnp.sum(jnp.where(sel2, rank + base_row, 0.0), axis=1, keepdims=True)
    r1_ref[...] = r1v.astype(jnp.int32)
    r2_ref[...] = r2v.astype(jnp.int32)

    bio = lax.broadcasted_iota(jnp.int32, (E, NB), 1).astype(f32)
    bsafe = jnp.minimum(bio, total - 1.0)
    cmp = (pad_t <= bsafe).astype(f32)
    be_row = jnp.sum(cmp, axis=0, keepdims=True) - 1.0
    bio1 = lax.broadcasted_iota(jnp.int32, (1, NB), 1).astype(f32)
    be_ref[...] = be_row.astype(jnp.int32)
    bv_ref[...] = (bio1 < total).astype(jnp.int32)


def _sc_dispatch_kernel(S, Hd, CH, NW, x_hbm, r1_hbm, r2_hbm, xs_hbm,
                        idx_v, idx2_v, rows_v, sem):
    nc = lax.axis_index("c")
    ns = lax.axis_index("s")
    wid = ns * 2 + nc
    per_w = S // NW
    nch = per_w // CH

    def chunk_body(c, carry):
        base = wid * per_w + c * CH
        pltpu.sync_copy(x_hbm.at[pl.ds(base, CH)], rows_v)
        pltpu.sync_copy(r1_hbm.at[pl.ds(base, CH)], idx_v)
        pltpu.sync_copy(r2_hbm.at[pl.ds(base, CH)], idx2_v)
        c1 = pltpu.async_copy(rows_v, xs_hbm.at[idx_v], sem)
        c2 = pltpu.async_copy(rows_v, xs_hbm.at[idx2_v], sem)
        c1.wait()
        c2.wait()
        return carry

    lax.fori_loop(0, nch, chunk_body, 0, unroll=False)


def _ffn_kernel(T, KI, be_ref, bv_ref,
                xs_in_ref, w1_ref, b1_ref, w2_ref, b2_ref,
                y_ref, xs_ref, ya_ref):
    f32 = jnp.float32
    bf16 = jnp.bfloat16
    b = pl.program_id(0)
    ki = pl.program_id(1)

    valid = bv_ref[b] == 1

    @pl.when(valid)
    def _body():
        @pl.when(ki == 0)
        def _stage():
            xs_ref[...] = xs_in_ref[...].astype(bf16)

        h = lax.dot(xs_ref[...], w1_ref[0].astype(bf16),
                    preferred_element_type=f32) + b1_ref[0]
        h = jax.nn.gelu(h, approximate=True)
        part = lax.dot(h.astype(bf16), w2_ref[0].astype(bf16),
                       preferred_element_type=f32)

        @pl.when(ki == 0)
        def _acc0():
            ya_ref[...] = part

        @pl.when(ki != 0)
        def _accn():
            ya_ref[...] += part

        @pl.when(ki == KI - 1)
        def _emit():
            y_ref[...] = ya_ref[...] + b2_ref[0]


def _sc_combine_kernel(S, Hd, CH, NW, y_hbm, r1_hbm, r2_hbm, w1_hbm, w2_hbm,
                       out_hbm, idx_v, idx2_v, wv1_v, wv2_v,
                       rows1_v, rows2_v, sem):
    nc = lax.axis_index("c")
    ns = lax.axis_index("s")
    wid = ns * 2 + nc
    per_w = S // NW
    nch = per_w // CH

    def chunk_body(c, carry):
        base = wid * per_w + c * CH
        pltpu.sync_copy(r1_hbm.at[pl.ds(base, CH)], idx_v)
        pltpu.sync_copy(r2_hbm.at[pl.ds(base, CH)], idx2_v)
        pltpu.sync_copy(w1_hbm.at[pl.ds(base, CH)], wv1_v)
        pltpu.sync_copy(w2_hbm.at[pl.ds(base, CH)], wv2_v)
        c1 = pltpu.async_copy(y_hbm.at[idx_v], rows1_v, sem)
        c2 = pltpu.async_copy(y_hbm.at[idx2_v], rows2_v, sem)
        c1.wait()
        c2.wait()

        def row_body(i, carry2):
            w1b = wv1_v[i, :]
            w2b = wv2_v[i, :]
            for j in range(Hd // 16):
                sl = pl.ds(j * 16, 16)
                rows1_v[i, sl] = rows1_v[i, sl] * w1b + rows2_v[i, sl] * w2b
            return carry2

        lax.fori_loop(0, CH, row_body, 0, unroll=False)
        pltpu.sync_copy(rows1_v, out_hbm.at[pl.ds(base, CH)])
        return carry

    lax.fori_loop(0, nch, chunk_body, 0, unroll=False)


def kernel(x, gate_w, gate_b, w1, b1, w2, b2):
    Bx, Sx, Hd = x.shape
    E = gate_w.shape[1]
    I = w1.shape[2]
    S = Bx * Sx
    T = 576
    NB = -((-2 * S) // T) + (E - 1)
    TI = 1024
    KI = I // TI
    NW = 32
    CH = 32

    flat32 = x.reshape(S, Hd)
    flat = flat32.astype(jnp.bfloat16)
    gb2 = gate_b.reshape(1, E)

    rank, be, bv, r1, r2, wn1, wn2 = pl.pallas_call(
        functools.partial(_router_kernel, T, NB),
        out_shape=[
            jax.ShapeDtypeStruct((S, E), jnp.float32),
            jax.ShapeDtypeStruct((1, NB), jnp.int32),
            jax.ShapeDtypeStruct((1, NB), jnp.int32),
            jax.ShapeDtypeStruct((S, 1), jnp.int32),
            jax.ShapeDtypeStruct((S, 1), jnp.int32),
            jax.ShapeDtypeStruct((S, 16), jnp.float32),
            jax.ShapeDtypeStruct((S, 16), jnp.float32),
        ],
    )(flat, gate_w, gb2)

    be = be.reshape(NB)
    bv = bv.reshape(NB)
    r1f = r1.reshape(S)
    r2f = r2.reshape(S)

    mesh = plsc.VectorSubcoreMesh(core_axis_name="c", subcore_axis_name="s")
    xs_all = pl.kernel(
        functools.partial(_sc_dispatch_kernel, S, Hd, CH, NW),
        out_type=jax.ShapeDtypeStruct((NB * T, Hd), jnp.float32),
        mesh=mesh,
        scratch_types=[
            pltpu.VMEM((CH,), jnp.int32),
            pltpu.VMEM((CH,), jnp.int32),
            pltpu.VMEM((CH, Hd), jnp.float32),
            pltpu.SemaphoreType.DMA,
        ],
    )(flat32, r1f, r2f)

    def _clamped_ki(b_i, ki_i, bv_s):
        return jnp.where(bv_s[b_i] == 1, ki_i, KI - 1)

    def w1_map(b_i, ki_i, be_s, bv_s):
        return (be_s[b_i], 0, _clamped_ki(b_i, ki_i, bv_s))

    def b1_map(b_i, ki_i, be_s, bv_s):
        return (be_s[b_i] * KI + _clamped_ki(b_i, ki_i, bv_s), 0, 0)

    def w2_map(b_i, ki_i, be_s, bv_s):
        return (be_s[b_i], _clamped_ki(b_i, ki_i, bv_s), 0)

    def b2_map(b_i, ki_i, be_s, bv_s):
        return (be_s[b_i], 0, 0)

    grid_spec = pltpu.PrefetchScalarGridSpec(
        num_scalar_prefetch=2,
        grid=(NB, KI),
        in_specs=[
            pl.BlockSpec((T, Hd), lambda b_i, ki_i, *_: (b_i, 0)),
            pl.BlockSpec((1, Hd, TI), w1_map),
            pl.BlockSpec((1, 1, TI), b1_map),
            pl.BlockSpec((1, TI, Hd), w2_map),
            pl.BlockSpec((1, 1, Hd), b2_map),
        ],
        out_specs=pl.BlockSpec((T, Hd), lambda b_i, ki_i, *_: (b_i, 0)),
        scratch_shapes=[
            pltpu.VMEM((T, Hd), jnp.bfloat16),
            pltpu.VMEM((T, Hd), jnp.float32),
        ],
    )

    y_all = pl.pallas_call(
        functools.partial(_ffn_kernel, T, KI),
        grid_spec=grid_spec,
        out_shape=jax.ShapeDtypeStruct((NB * T, Hd), jnp.float32),
    )(be, bv, xs_all,
      w1, b1.reshape(E * KI, 1, TI), w2, b2.reshape(E, 1, Hd))

    out = pl.kernel(
        functools.partial(_sc_combine_kernel, S, Hd, CH, NW),
        out_type=jax.ShapeDtypeStruct((S, Hd), jnp.float32),
        mesh=mesh,
        scratch_types=[
            pltpu.VMEM((CH,), jnp.int32),
            pltpu.VMEM((CH,), jnp.int32),
            pltpu.VMEM((CH, 16), jnp.float32),
            pltpu.VMEM((CH, 16), jnp.float32),
            pltpu.VMEM((CH, Hd), jnp.float32),
            pltpu.VMEM((CH, Hd), jnp.float32),
            pltpu.SemaphoreType.DMA,
        ],
    )(y_all, r1f, r2f, wn1, wn2)

    return out.reshape(Bx, Sx, Hd)


# speedup vs baseline: 1.1358x; 1.1358x over previous
import functools

import jax
import jax.numpy as jnp
from jax import lax
from jax.experimental import pallas as pl
from jax.experimental.pallas import tpu as pltpu


def _router_kernel(T, NB, x_ref, gw_ref, gb_ref,
                   rank_ref, wc_ref, be_ref, bs_ref, bv_ref):
    S, Hd = x_ref.shape
    E = gw_ref.shape[1]
    f32 = jnp.float32
    bf16 = jnp.bfloat16

    logits = lax.dot(x_ref[...], gw_ref[...].astype(bf16),
                     preferred_element_type=f32) + gb_ref[...]

    m1 = jnp.max(logits, axis=1, keepdims=True)
    ex = jnp.exp(logits - m1)
    z = jnp.sum(ex, axis=1, keepdims=True)
    p = ex / z
    ioe = lax.broadcasted_iota(jnp.int32, (S, E), 1)
    pm1 = jnp.max(p, axis=1, keepdims=True)
    i1 = jnp.min(jnp.where(p == pm1, ioe, E), axis=1, keepdims=True)
    p_m = jnp.where(ioe == i1, -1.0, p)
    pm2 = jnp.max(p_m, axis=1, keepdims=True)
    i2 = jnp.min(jnp.where(p_m == pm2, ioe, E), axis=1, keepdims=True)
    denom = pm1 + pm2 + 1e-6
    sel1 = ioe == i1
    sel2 = ioe == i2
    wc = jnp.where(sel1, pm1 / denom, 0.0) + jnp.where(sel2, pm2 / denom, 0.0)
    m = jnp.logical_or(sel1, sel2).astype(f32)

    rio = lax.broadcasted_iota(jnp.int32, (S, S), 0)
    cio = lax.broadcasted_iota(jnp.int32, (S, S), 1)
    ltri = (cio < rio).astype(bf16)
    rank = lax.dot(ltri, m.astype(bf16), preferred_element_type=f32)
    rank_ref[...] = jnp.where(m > 0.5, rank, -1.0)
    wc_ref[...] = wc

    ones_col = jnp.ones((S, 1), bf16)
    counts_t = lax.dot_general(m.astype(bf16), ones_col,
                               (((0,), (0,)), ((), ())),
                               preferred_element_type=f32)
    nb_t = jnp.floor((counts_t + (T - 1)) / T)
    eio_r = lax.broadcasted_iota(jnp.int32, (E, E), 0)
    eio_c = lax.broadcasted_iota(jnp.int32, (E, E), 1)
    l8 = (eio_r > eio_c).astype(bf16)
    pad_t = lax.dot(l8, nb_t.astype(bf16), preferred_element_type=f32)
    total = jnp.sum(nb_t, axis=0, keepdims=True)

    bio = lax.broadcasted_iota(jnp.int32, (E, NB), 1).astype(f32)
    bsafe = jnp.minimum(bio, total - 1.0)
    cmp = (pad_t <= bsafe).astype(f32)
    be_row = jnp.sum(cmp, axis=0, keepdims=True) - 1.0
    pad_sel = jnp.max(jnp.where(cmp > 0.5, jnp.broadcast_to(pad_t, (E, NB)), 0.0),
                      axis=0, keepdims=True)
    bio1 = lax.broadcasted_iota(jnp.int32, (1, NB), 1).astype(f32)
    bsafe1 = jnp.minimum(bio1, total - 1.0)
    bs_row = (bsafe1 - pad_sel) * T
    be_ref[...] = be_row.astype(jnp.int32)
    bs_ref[...] = bs_row.astype(jnp.int32)
    bv_ref[...] = (bio1 < total).astype(jnp.int32)


def _ffn_kernel(T, KI, be_ref, bs_ref, bv_ref,
                x_ref, rank_ref, wc_ref, w1_ref, b1_ref, w2_ref, b2_ref,
                out_ref, pt_ref, xs_ref, ya_ref):
    S, Hd = x_ref.shape
    E = rank_ref.shape[1]
    f32 = jnp.float32
    bf16 = jnp.bfloat16
    b = pl.program_id(0)
    ki = pl.program_id(1)

    @pl.when(jnp.logical_and(b == 0, ki == 0))
    def _init():
        out_ref[...] = jnp.zeros_like(out_ref)

    valid = bv_ref[b] == 1
    e = be_ref[b]

    @pl.when(valid)
    def _body():
        @pl.when(ki == 0)
        def _gather():
            ioe = lax.broadcasted_iota(jnp.int32, (S, E), 1)
            sel = ioe == e
            r = jnp.sum(jnp.where(sel, rank_ref[...], 0.0), axis=1, keepdims=True)
            tio = lax.broadcasted_iota(jnp.int32, (S, T), 1).astype(f32)
            startf = bs_ref[b].astype(f32)
            pt_ref[...] = (r == tio + startf).astype(bf16)
            xs = lax.dot_general(pt_ref[...], x_ref[...],
                                 (((0,), (0,)), ((), ())),
                                 preferred_element_type=f32)
            xs_ref[...] = xs.astype(bf16)

        h = lax.dot(xs_ref[...], w1_ref[0].astype(bf16),
                    preferred_element_type=f32) + b1_ref[0]
        h = jax.nn.gelu(h.astype(bf16), approximate=True)
        part = lax.dot(h, w2_ref[0].astype(bf16),
                       preferred_element_type=f32)

        @pl.when(ki == 0)
        def _acc0():
            ya_ref[...] = part

        @pl.when(ki != 0)
        def _accn():
            ya_ref[...] += part

        @pl.when(ki == KI - 1)
        def _combine():
            y = (ya_ref[...] + b2_ref[0]).astype(bf16)
            res = lax.dot(pt_ref[...], y, preferred_element_type=f32)
            ioe = lax.broadcasted_iota(jnp.int32, (S, E), 1)
            wv = jnp.sum(jnp.where(ioe == e, wc_ref[...], 0.0),
                         axis=1, keepdims=True)
            out_ref[...] += wv * res


def kernel(x, gate_w, gate_b, w1, b1, w2, b2):
    Bx, Sx, Hd = x.shape
    E = gate_w.shape[1]
    I = w1.shape[2]
    S = Bx * Sx
    T = 576
    NB = -((-2 * S) // T) + (E - 1)
    TI = 1024
    KI = I // TI

    flat = x.reshape(S, Hd).astype(jnp.bfloat16)
    gb2 = gate_b.reshape(1, E)

    rank, wc, be, bs, bv = pl.pallas_call(
        functools.partial(_router_kernel, T, NB),
        out_shape=[
            jax.ShapeDtypeStruct((S, E), jnp.float32),
            jax.ShapeDtypeStruct((S, E), jnp.float32),
            jax.ShapeDtypeStruct((1, NB), jnp.int32),
            jax.ShapeDtypeStruct((1, NB), jnp.int32),
            jax.ShapeDtypeStruct((1, NB), jnp.int32),
        ],
    )(flat, gate_w, gb2)

    be = be.reshape(NB)
    bs = bs.reshape(NB)
    bv = bv.reshape(NB)

    def _clamped_ki(b_i, ki_i, bv_s):
        return jnp.where(bv_s[b_i] == 1, ki_i, KI - 1)

    def w1_map(b_i, ki_i, be_s, bs_s, bv_s):
        return (be_s[b_i], 0, _clamped_ki(b_i, ki_i, bv_s))

    def b1_map(b_i, ki_i, be_s, bs_s, bv_s):
        return (be_s[b_i] * KI + _clamped_ki(b_i, ki_i, bv_s), 0, 0)

    def w2_map(b_i, ki_i, be_s, bs_s, bv_s):
        return (be_s[b_i], _clamped_ki(b_i, ki_i, bv_s), 0)

    def b2_map(b_i, ki_i, be_s, bs_s, bv_s):
        return (be_s[b_i], 0, 0)

    grid_spec = pltpu.PrefetchScalarGridSpec(
        num_scalar_prefetch=3,
        grid=(NB, KI),
        in_specs=[
            pl.BlockSpec((S, Hd), lambda b_i, ki_i, *_: (0, 0)),
            pl.BlockSpec((S, E), lambda b_i, ki_i, *_: (0, 0)),
            pl.BlockSpec((S, E), lambda b_i, ki_i, *_: (0, 0)),
            pl.BlockSpec((1, Hd, TI), w1_map),
            pl.BlockSpec((1, 1, TI), b1_map),
            pl.BlockSpec((1, TI, Hd), w2_map),
            pl.BlockSpec((1, 1, Hd), b2_map),
        ],
        out_specs=pl.BlockSpec((S, Hd), lambda b_i, ki_i, *_: (0, 0)),
        scratch_shapes=[
            pltpu.VMEM((S, T), jnp.bfloat16),
            pltpu.VMEM((T, Hd), jnp.bfloat16),
            pltpu.VMEM((T, Hd), jnp.float32),
        ],
    )

    out = pl.pallas_call(
        functools.partial(_ffn_kernel, T, KI),
        grid_spec=grid_spec,
        out_shape=jax.ShapeDtypeStruct((S, Hd), jnp.float32),
    )(be, bs, bv, flat, rank, wc,
      w1, b1.reshape(E * KI, 1, TI), w2, b2.reshape(E, 1, Hd))

    return out.reshape(Bx, Sx, Hd)
